# baseline (speedup 1.0000x reference)
import jax
import jax.numpy as jnp
from jax import lax
from jax.experimental import pallas as pl
from jax.experimental.pallas import tpu as pltpu

BATCH = 4
CTX = 8192
EMB = 1024
CR = 256
D = 3
NCH = CTX // CR


def _body(x_hbm, pos_hbm, o_hbm, xb, pb, six, sip, so):
    def issue_in(j):
        d = lax.rem(j, D)
        s0 = j * CR
        pltpu.make_async_copy(pos_hbm.at[pl.ds(s0, CR), :], pb.at[d],
                              sip.at[d]).start()
        for b in range(BATCH):
            pltpu.make_async_copy(x_hbm.at[b, pl.ds(s0, CR), :], xb.at[d, b],
                                  six.at[d]).start()

    for j in range(D):
        issue_in(j)

    @pl.loop(0, NCH)
    def _(k):
        d = lax.rem(k, D)

        @pl.when(k >= 1)
        def _():
            dprev = lax.rem(k + (D - 1), D)
            for b in range(BATCH):
                pltpu.make_async_copy(xb.at[dprev, b],
                                      o_hbm.at[b, pl.ds(0, CR), :],
                                      so.at[dprev]).wait()

            @pl.when(k - 1 + D < NCH)
            def _():
                issue_in(k - 1 + D)

        pltpu.make_async_copy(pos_hbm.at[pl.ds(0, CR), :], pb.at[d],
                              sip.at[d]).wait()
        for b in range(BATCH):
            pltpu.make_async_copy(x_hbm.at[0, pl.ds(0, CR), :], xb.at[d, b],
                                  six.at[d]).wait()

        for b in range(BATCH):
            xb[d, b] = xb[d, b] + pb[d]

        for b in range(BATCH):
            pltpu.make_async_copy(xb.at[d, b],
                                  o_hbm.at[b, pl.ds(k * CR, CR), :],
                                  so.at[d]).start()

    dlast = (NCH - 1) % D
    for b in range(BATCH):
        pltpu.make_async_copy(xb.at[dlast, b], o_hbm.at[b, pl.ds(0, CR), :],
                              so.at[dlast]).wait()


def kernel(x, pos_table):
    return pl.pallas_call(
        _body,
        in_specs=[
            pl.BlockSpec(memory_space=pl.ANY),
            pl.BlockSpec(memory_space=pl.ANY),
        ],
        out_specs=pl.BlockSpec(memory_space=pl.ANY),
        out_shape=jax.ShapeDtypeStruct(x.shape, x.dtype),
        scratch_shapes=[
            pltpu.VMEM((D, BATCH, CR, EMB), jnp.float32),
            pltpu.VMEM((D, CR, EMB), jnp.float32),
            pltpu.SemaphoreType.DMA((D,)),
            pltpu.SemaphoreType.DMA((D,)),
            pltpu.SemaphoreType.DMA((D,)),
        ],
    )(x, pos_table)


# speedup vs baseline: 1.0031x; 1.0031x over previous
import jax
import jax.numpy as jnp
from jax import lax
from jax.experimental import pallas as pl
from jax.experimental.pallas import tpu as pltpu

BATCH = 4
CTX = 8192
EMB = 1024
CR = 512
D = 4
NCH = CTX // CR


def _body(x_hbm, pos_hbm, o_hbm, xb, pb, six, sip, so):
    def issue_in(j):
        d = lax.rem(j, D)
        s0 = j * CR
        pltpu.make_async_copy(pos_hbm.at[pl.ds(s0, CR), :], pb.at[d],
                              sip.at[d]).start()
        for b in range(BATCH):
            pltpu.make_async_copy(x_hbm.at[b, pl.ds(s0, CR), :], xb.at[d, b],
                                  six.at[d]).start()

    for j in range(D):
        issue_in(j)

    @pl.loop(0, NCH)
    def _(k):
        d = lax.rem(k, D)

        @pl.when(k >= 1)
        def _():
            dprev = lax.rem(k + (D - 1), D)
            for b in range(BATCH):
                pltpu.make_async_copy(xb.at[dprev, b],
                                      o_hbm.at[b, pl.ds(0, CR), :],
                                      so.at[dprev]).wait()

            @pl.when(k - 1 + D < NCH)
            def _():
                issue_in(k - 1 + D)

        pltpu.make_async_copy(pos_hbm.at[pl.ds(0, CR), :], pb.at[d],
                              sip.at[d]).wait()
        for b in range(BATCH):
            pltpu.make_async_copy(x_hbm.at[0, pl.ds(0, CR), :], xb.at[d, b],
                                  six.at[d]).wait()

        for b in range(BATCH):
            xb[d, b] = xb[d, b] + pb[d]

        for b in range(BATCH):
            pltpu.make_async_copy(xb.at[d, b],
                                  o_hbm.at[b, pl.ds(k * CR, CR), :],
                                  so.at[d]).start()

    dlast = (NCH - 1) % D
    for b in range(BATCH):
        pltpu.make_async_copy(xb.at[dlast, b], o_hbm.at[b, pl.ds(0, CR), :],
                              so.at[dlast]).wait()


def kernel(x, pos_table):
    return pl.pallas_call(
        _body,
        in_specs=[
            pl.BlockSpec(memory_space=pl.ANY),
            pl.BlockSpec(memory_space=pl.ANY),
        ],
        out_specs=pl.BlockSpec(memory_space=pl.ANY),
        out_shape=jax.ShapeDtypeStruct(x.shape, x.dtype),
        scratch_shapes=[
            pltpu.VMEM((D, BATCH, CR, EMB), jnp.float32),
            pltpu.VMEM((D, CR, EMB), jnp.float32),
            pltpu.SemaphoreType.DMA((D,)),
            pltpu.SemaphoreType.DMA((D,)),
            pltpu.SemaphoreType.DMA((D,)),
        ],
    )(x, pos_table)


# speedup vs baseline: 1.0315x; 1.0284x over previous
import jax
import jax.numpy as jnp
from jax.experimental import pallas as pl

BLK = 512


def _add_kernel(x_ref, pos_ref, o_ref):
    o_ref[...] = x_ref[...] + pos_ref[...][None, :, :]


def kernel(x, pos_table):
    batch, ctx, emb = x.shape
    grid = (ctx // BLK,)
    return pl.pallas_call(
        _add_kernel,
        grid=grid,
        in_specs=[
            pl.BlockSpec((batch, BLK, emb), lambda i: (0, i, 0)),
            pl.BlockSpec((BLK, emb), lambda i: (i, 0)),
        ],
        out_specs=pl.BlockSpec((batch, BLK, emb), lambda i: (0, i, 0)),
        out_shape=jax.ShapeDtypeStruct(x.shape, x.dtype),
    )(x, pos_table)
